# Initial kernel scaffold; baseline (speedup 1.0000x reference)
import jax
import jax.numpy as jnp
from jax.experimental import pallas as pl


def kernel(x, edge_index, edge_attr, batch, cond, params):
    raise NotImplementedError("write your pallas kernel here")


# speedup vs baseline: 8.7805x; 8.7805x over previous
import functools
import jax
import jax.numpy as jnp
import numpy as np
from jax import lax
from jax.experimental import pallas as pl
from jax.experimental.pallas import tpu as pltpu, tpu_sc as plsc

N = 50000
E = 800000
G = 128
XD = 128
ED = 16
GD = 32
H = 64
NH = 2
EPS = 1e-5

M = N + G
Mp = 50176
HM = Mp // 2
Ea = E + 2 * N + M
CH = 128
NS = 16
CHGRP = NS * CH * 2
Eap = ((Ea + CHGRP - 1) // CHGRP) * CHGRP
E2n = E + 2 * N
E2np = ((E2n + CHGRP - 1) // CHGRP) * CHGRP
Q4 = Mp // 4
R16 = Mp // 16
NU = 272
Np = 50048
BM = 256

_mesh = plsc.VectorSubcoreMesh(core_axis_name="c", subcore_axis_name="s")
_sc_params = pltpu.CompilerParams(use_tc_tiling_on_sc=False)


def _wid():
  return lax.axis_index("s") * 2 + lax.axis_index("c")


def _zero_buf(buf, rows, width):
  def zrow(r, _):
    for j in range(width // 16):
      buf[r, pl.ds(j * 16, 16)] = jnp.zeros((16,), jnp.float32)
    return 0
  lax.fori_loop(0, rows, zrow, 0)


def _zero_acc_stripe(acc, buf, s, half_rows, strip, nstrip, width):
  _zero_buf(buf, strip, width)
  per = half_rows // NS
  for t in range(nstrip):
    pltpu.sync_copy(buf, acc.at[pl.ds(s * per + t * strip, strip)])

  @pl.when(s == 0)
  def _():
    pltpu.sync_copy(buf.at[pl.ds(0, 8)], acc.at[pl.ds(half_rows, 8)])


def _local_idx(dst_v, dstl_v, base, nrows):
  for j in range(CH // 16):
    d = dst_v[pl.ds(j * 16, 16)]
    l = d - base
    oob = (l < 0) | (l >= nrows)
    dstl_v[pl.ds(j * 16, 16)] = jnp.where(oob, nrows, l)


@functools.partial(
    pl.kernel,
    out_type=jax.ShapeDtypeStruct((Mp, H), jnp.float32),
    mesh=_mesh, compiler_params=_sc_params,
    scratch_types=[
        pltpu.VMEM((CH,), jnp.int32),
        pltpu.VMEM((CH,), jnp.int32),
        pltpu.VMEM((CH, H), jnp.float32),
        pltpu.VMEM((112, H), jnp.float32),
        pltpu.VMEM_SHARED((HM + 8, H), jnp.float32),
        pltpu.SemaphoreType.DMA,
    ],
)
def _sc_loop_attr(ae0, dst0, out, dst_v, dstl_v, ae_v, buf_v, acc, sem):
  del sem
  c = lax.axis_index("c")
  s = lax.axis_index("s")
  base = c * HM
  _zero_acc_stripe(acc, buf_v, s, HM, 112, 14, H)
  plsc.subcore_barrier()

  def body(k, _):
    eb = (k * NS + s) * CH
    pltpu.sync_copy(dst0.at[pl.ds(eb, CH)], dst_v)
    pltpu.sync_copy(ae0.at[pl.ds(eb, CH)], ae_v)
    _local_idx(dst_v, dstl_v, base, HM)
    pltpu.sync_copy(ae_v, acc.at[dstl_v], add=True)
    return 0

  lax.fori_loop(0, E2np // CHGRP * 2, body, 0)
  plsc.subcore_barrier()
  per = HM // NS
  for t in range(14):
    pltpu.sync_copy(acc.at[pl.ds(s * per + t * 112, 112)], buf_v)
    pltpu.sync_copy(buf_v, out.at[pl.ds(base + s * per + t * 112, 112)])


@functools.partial(
    pl.kernel,
    out_type=jax.ShapeDtypeStruct((Eap, H), jnp.float32),
    mesh=_mesh, compiler_params=_sc_params,
    scratch_types=[
        pltpu.VMEM((CH,), jnp.int32),
        pltpu.VMEM((CH, H), jnp.float32),
        pltpu.SemaphoreType.DMA,
    ],
)
def _sc_permute_ae(ae_full, perm, out, idx_v, rows_v, sem):
  c = lax.axis_index("c")
  s = lax.axis_index("s")

  def body(k, _):
    eb = c * (Eap // 2) + (k * NS + s) * CH
    pltpu.sync_copy(perm.at[pl.ds(eb, CH)], idx_v)
    pltpu.async_copy(ae_full.at[idx_v], rows_v, sem).wait()
    pltpu.sync_copy(rows_v, out.at[pl.ds(eb, CH)])
    return 0

  lax.fori_loop(0, Eap // CHGRP, body, 0)


@functools.partial(
    pl.kernel,
    out_type=jax.ShapeDtypeStruct((Mp, H), jnp.float32),
    mesh=_mesh, compiler_params=_sc_params,
    scratch_types=[
        pltpu.VMEM((16,), jnp.int32),
        pltpu.VMEM((CH,), jnp.int32),
        pltpu.VMEM((CH,), jnp.int32),
        pltpu.VMEM((CH,), jnp.int32),
        pltpu.VMEM((CH, H), jnp.float32),
        pltpu.VMEM((CH, H), jnp.float32),
        pltpu.VMEM((CH, H), jnp.float32),
        pltpu.VMEM((112, H), jnp.float32),
        pltpu.VMEM_SHARED((Q4 + 8, H), jnp.float32),
        pltpu.SemaphoreType.DMA,
    ],
)
def _sc_gen(on_t, ae_s, src_s, dst_s, bnds, out, bnds_v, src_v, dst_v, dstl_v,
            rows_v, ae_v, msg_v, buf_v, acc, sem):
  c = lax.axis_index("c")
  s = lax.axis_index("s")
  wid = _wid()
  for rr in range(2):
    rg = c * 2 + rr
    base = rg * Q4
    _zero_acc_stripe(acc, buf_v, s, Q4, 112, 7, H)
    plsc.subcore_barrier()

    pltpu.sync_copy(bnds.at[rg * 32 + wid], bnds_v)
    bv = bnds_v[...]
    e0 = bv[0]
    nk = bv[1]

    def body(k, _):
      eb = pl.multiple_of(e0 + (k * NS + s) * CH, 8)
      pltpu.sync_copy(src_s.at[pl.ds(eb, CH)], src_v)
      pltpu.sync_copy(dst_s.at[pl.ds(eb, CH)], dst_v)
      pltpu.async_copy(on_t.at[src_v], rows_v, sem).wait()
      pltpu.sync_copy(ae_s.at[pl.ds(eb, CH)], ae_v)

      def crow(r, _):
        for t in range(H // 16):
          v = rows_v[r, pl.ds(t * 16, 16)] + ae_v[r, pl.ds(t * 16, 16)]
          msg_v[r, pl.ds(t * 16, 16)] = jnp.maximum(v, 0.0) + 1e-7
        return 0
      lax.fori_loop(0, CH, crow, 0)

      _local_idx(dst_v, dstl_v, base, Q4)
      pltpu.sync_copy(msg_v, acc.at[dstl_v], add=True)
      return 0

    lax.fori_loop(0, nk, body, 0)
    plsc.subcore_barrier()
    per = Q4 // NS
    for t in range(7):
      pltpu.sync_copy(acc.at[pl.ds(s * per + t * 112, 112)], buf_v)
      pltpu.sync_copy(buf_v, out.at[pl.ds(base + s * per + t * 112, 112)])
    plsc.subcore_barrier()


@functools.partial(
    pl.kernel,
    out_type=[jax.ShapeDtypeStruct((Eap, 4 * H), jnp.float32),
              jax.ShapeDtypeStruct((Eap, 2 * H), jnp.float32)],
    mesh=_mesh, compiler_params=_sc_params,
    scratch_types=[
        pltpu.VMEM((CH,), jnp.int32),
        pltpu.VMEM((CH,), jnp.int32),
        pltpu.VMEM((CH,), jnp.int32),
        pltpu.VMEM((CH, 4 * H), jnp.float32),
        pltpu.VMEM((CH, 2 * H), jnp.float32),
        pltpu.SemaphoreType.DMA,
        pltpu.SemaphoreType.DMA,
    ],
)
def _sc_gather_qk(qu_t, k_t, src_s, dst_s, qd_out, ks_out, src_v, dst_v,
                  dstc_v, qu_v, k_v, sem1, sem2):
  c = lax.axis_index("c")
  s = lax.axis_index("s")

  def body(k, _):
    eb = c * (Eap // 2) + (k * NS + s) * CH
    pltpu.sync_copy(src_s.at[pl.ds(eb, CH)], src_v)
    pltpu.sync_copy(dst_s.at[pl.ds(eb, CH)], dst_v)
    for j in range(CH // 16):
      d = dst_v[pl.ds(j * 16, 16)]
      dstc_v[pl.ds(j * 16, 16)] = jnp.where(d >= Mp, 0, d)
    cp1 = pltpu.async_copy(qu_t.at[dstc_v], qu_v, sem1)
    cp2 = pltpu.async_copy(k_t.at[src_v], k_v, sem2)
    cp1.wait()
    cp2.wait()
    pltpu.sync_copy(qu_v, qd_out.at[pl.ds(eb, CH)])
    pltpu.sync_copy(k_v, ks_out.at[pl.ds(eb, CH)])
    return 0

  lax.fori_loop(0, Eap // CHGRP, body, 0)


BE = 512


def _logits_call(qd, ks, ae):
  def body(qd_ref, ks_ref, ae_ref, z_ref):
    qd_b = qd_ref[...]
    ks_b = ks_ref[...]
    ae_b = ae_ref[...]
    l0 = (jnp.sum(qd_b[:, :H] * ks_b[:, :H], axis=1, keepdims=True)
          + jnp.sum(qd_b[:, 2 * H:3 * H] * ae_b, axis=1, keepdims=True))
    l1 = (jnp.sum(qd_b[:, H:2 * H] * ks_b[:, H:], axis=1, keepdims=True)
          + jnp.sum(qd_b[:, 3 * H:] * ae_b, axis=1, keepdims=True))
    ones = jnp.ones((1, 16), jnp.float32)
    z_ref[...] = jnp.concatenate(
        [jnp.exp(l0 * 0.125) * ones, jnp.exp(l1 * 0.125) * ones], 1)

  return pl.pallas_call(
      body,
      grid=(Eap // BE,),
      in_specs=[pl.BlockSpec((BE, 4 * H), lambda i: (i, 0)),
                pl.BlockSpec((BE, 2 * H), lambda i: (i, 0)),
                pl.BlockSpec((BE, H), lambda i: (i, 0))],
      out_specs=pl.BlockSpec((BE, 32), lambda i: (i, 0)),
      out_shape=jax.ShapeDtypeStruct((Eap, 32), jnp.float32),
  )(qd, ks, ae)


@functools.partial(
    pl.kernel,
    out_type=jax.ShapeDtypeStruct((Mp, NU), jnp.float32),
    mesh=_mesh, compiler_params=_sc_params,
    scratch_types=[
        pltpu.VMEM((16,), jnp.int32),
        pltpu.VMEM((CH,), jnp.int32),
        pltpu.VMEM((CH,), jnp.int32),
        pltpu.VMEM((CH,), jnp.int32),
        pltpu.VMEM((CH, 2 * H), jnp.float32),
        pltpu.VMEM((CH, H), jnp.float32),
        pltpu.VMEM((CH, 32), jnp.float32),
        pltpu.VMEM((CH, NU), jnp.float32),
        pltpu.VMEM((28, NU), jnp.float32),
        pltpu.VMEM_SHARED((R16 + 8, NU), jnp.float32),
        pltpu.SemaphoreType.DMA,
    ],
)
def _sc_numer(v_t, ae_s, z, src_s, dst_s, bnds, out, bnds_v, src_v, dst_v,
              dstl_v, v_v, ae_v, z_v, row_v, buf_v, acc, sem):
  c = lax.axis_index("c")
  s = lax.axis_index("s")
  lanes = lax.iota(jnp.int32, 16)
  wid = _wid()

  for rr in range(8):
    rg = c * 8 + rr
    base = rg * R16
    _zero_acc_stripe(acc, buf_v, s, R16, 28, 7, NU)
    plsc.subcore_barrier()

    pltpu.sync_copy(bnds.at[rg * 32 + wid], bnds_v)
    bv = bnds_v[...]
    e0 = bv[0]
    nk = bv[1]

    def body(k, _):
      eb = pl.multiple_of(e0 + (k * NS + s) * CH, 8)
      pltpu.sync_copy(src_s.at[pl.ds(eb, CH)], src_v)
      pltpu.sync_copy(dst_s.at[pl.ds(eb, CH)], dst_v)
      pltpu.async_copy(v_t.at[src_v], v_v, sem).wait()
      pltpu.sync_copy(ae_s.at[pl.ds(eb, CH)], ae_v)
      pltpu.sync_copy(z.at[pl.ds(eb, CH)], z_v)

      def crow(r, _):
        zb0 = z_v[r, pl.ds(0, 16)]
        zb1 = z_v[r, pl.ds(16, 16)]
        for t in range(H // 16):
          row_v[r, pl.ds(t * 16, 16)] = v_v[r, pl.ds(t * 16, 16)] * zb0
          row_v[r, pl.ds(64 + t * 16, 16)] = v_v[r, pl.ds(64 + t * 16, 16)] * zb1
          a = ae_v[r, pl.ds(t * 16, 16)]
          row_v[r, pl.ds(128 + t * 16, 16)] = a * zb0
          row_v[r, pl.ds(192 + t * 16, 16)] = a * zb1
        row_v[r, pl.ds(256, 16)] = jnp.where(
            lanes == 0, zb0, jnp.where(lanes == 1, zb1, 0.0))
        return 0

      lax.fori_loop(0, CH, crow, 0)
      _local_idx(dst_v, dstl_v, base, R16)
      pltpu.sync_copy(row_v, acc.at[dstl_v], add=True)
      return 0

    lax.fori_loop(0, nk, body, 0)
    plsc.subcore_barrier()
    per = R16 // NS
    for t in range(7):
      pltpu.sync_copy(acc.at[pl.ds(s * per + t * 28, 28)], buf_v)
      pltpu.sync_copy(buf_v, out.at[pl.ds(base + s * per + t * 28, 28)])
    plsc.subcore_barrier()


def _lrelu(x):
  return jnp.where(x >= 0, x, 0.01 * x)


def _mlp3_call(x, ps, rows):
  din = x.shape[1]

  def body(x_ref, w1, b1, w2, b2, w3, b3, o_ref):
    h = _lrelu(jnp.dot(x_ref[...], w1[...],
                       preferred_element_type=jnp.float32) + b1[...])
    h = _lrelu(jnp.dot(h, w2[...], preferred_element_type=jnp.float32) + b2[...])
    o_ref[...] = jnp.dot(h, w3[...], preferred_element_type=jnp.float32) + b3[...]

  full = lambda a, b: pl.BlockSpec((a, b), lambda i: (0, 0))
  return pl.pallas_call(
      body,
      grid=(rows // BM,),
      in_specs=[
          pl.BlockSpec((BM, din), lambda i: (i, 0)),
          full(din, H), full(1, H), full(H, H), full(1, H), full(H, H),
          full(1, H),
      ],
      out_specs=pl.BlockSpec((BM, H), lambda i: (i, 0)),
      out_shape=jax.ShapeDtypeStruct((rows, H), jnp.float32),
  )(x, ps[0]["w"], ps[0]["b"].reshape(1, H), ps[1]["w"],
    ps[1]["b"].reshape(1, H), ps[2]["w"], ps[2]["b"].reshape(1, H))


def _scale_rows_call(sums, inv):
  def body(s_ref, i_ref, o_ref):
    o_ref[...] = s_ref[...] * i_ref[...][:, 0:1]

  return pl.pallas_call(
      body,
      grid=(Mp // BM,),
      in_specs=[pl.BlockSpec((BM, H), lambda i: (i, 0)),
                pl.BlockSpec((BM, 8), lambda i: (i, 0))],
      out_specs=pl.BlockSpec((BM, H), lambda i: (i, 0)),
      out_shape=jax.ShapeDtypeStruct((Mp, H), jnp.float32),
  )(sums, inv)


def _stats_call(o, oh):
  def body(o_ref, oh_ref, s_ref):
    @pl.when(pl.program_id(0) == 0)
    def _():
      s_ref[...] = jnp.zeros_like(s_ref)
    x = o_ref[...]
    s1 = jnp.sum(x, axis=1, keepdims=True)
    s2 = jnp.sum(x * x, axis=1, keepdims=True)
    both = jnp.concatenate([s1, s2, jnp.zeros((BM, 14), jnp.float32)], 1)
    s_ref[...] += jnp.dot(oh_ref[...].T, both,
                          preferred_element_type=jnp.float32)

  return pl.pallas_call(
      body,
      grid=(Mp // BM,),
      in_specs=[pl.BlockSpec((BM, H), lambda i: (i, 0)),
                pl.BlockSpec((BM, G), lambda i: (i, 0))],
      out_specs=pl.BlockSpec((G, 16), lambda i: (0, 0)),
      out_shape=jax.ShapeDtypeStruct((G, 16), jnp.float32),
  )(o, oh)


def _pre_call(o, oh, stats, invn, c_h, wcs, bcs):
  def body(o_ref, oh_ref, st_ref, in_ref, c_ref, w_ref, b_ref, on_ref, cs_ref):
    st = st_ref[...]
    invn = in_ref[...][:, 0:1]
    mean = st[:, 0:1] * invn
    var = st[:, 1:2] * invn - mean * mean
    rstd = jax.lax.rsqrt(var + EPS)
    mv = jnp.concatenate([mean, rstd, jnp.zeros((G, 14), jnp.float32)], 1)
    rows = jnp.dot(oh_ref[...], mv, preferred_element_type=jnp.float32)
    on_ref[...] = (o_ref[...] - rows[:, 0:1]) * rows[:, 1:2]
    csc = jnp.dot(c_ref[...], w_ref[...],
                  preferred_element_type=jnp.float32) + b_ref[...]
    cs_ref[...] = jnp.dot(oh_ref[...], csc, preferred_element_type=jnp.float32)

  full = lambda a, b: pl.BlockSpec((a, b), lambda i: (0, 0))
  return pl.pallas_call(
      body,
      grid=(Mp // BM,),
      in_specs=[
          pl.BlockSpec((BM, H), lambda i: (i, 0)),
          pl.BlockSpec((BM, G), lambda i: (i, 0)),
          full(G, 16), full(G, 8), full(G, H), full(H, 2 * H), full(1, 2 * H),
      ],
      out_specs=[pl.BlockSpec((BM, H), lambda i: (i, 0)),
                 pl.BlockSpec((BM, 2 * H), lambda i: (i, 0))],
      out_shape=[jax.ShapeDtypeStruct((Mp, H), jnp.float32),
                 jax.ShapeDtypeStruct((Mp, 2 * H), jnp.float32)],
  )(o, oh, stats, invn, c_h, wcs, bcs)


def _qkv_call(on, agg, lp, wet):
  wg, bg = lp["gen_mlp"]["w"], lp["gen_mlp"]["b"].reshape(1, H)
  wq, bq = lp["q"]["w"], lp["q"]["b"].reshape(1, 2 * H)
  wk, bk = lp["k"]["w"], lp["k"]["b"].reshape(1, 2 * H)
  wv, bv = lp["v"]["w"], lp["v"]["b"].reshape(1, 2 * H)
  ws, bs = lp["skip"]["w"], lp["skip"]["b"].reshape(1, 2 * H)

  def body(on_ref, ag_ref, wg_r, bg_r, wq_r, bq_r, wk_r, bk_r, wv_r, bv_r,
           ws_r, bs_r, wet_r, qu_ref, k_ref, v_ref, sk_ref):
    on_b = on_ref[...]
    ag2 = jnp.dot(ag_ref[...] + on_b, wg_r[...],
                  preferred_element_type=jnp.float32) + bg_r[...]

    def two(w, b):
      return (jnp.dot(on_b, w[:H], preferred_element_type=jnp.float32)
              + jnp.dot(ag2, w[H:], preferred_element_type=jnp.float32) + b)

    q = two(wq_r[...], bq_r[...])
    u0 = jnp.dot(q[:, :H], wet_r[...][:H], preferred_element_type=jnp.float32)
    u1 = jnp.dot(q[:, H:], wet_r[...][H:], preferred_element_type=jnp.float32)
    qu_ref[...] = jnp.concatenate([q, u0, u1], 1)
    k_ref[...] = two(wk_r[...], bk_r[...])
    v_ref[...] = two(wv_r[...], bv_r[...])
    sk_ref[...] = two(ws_r[...], bs_r[...])

  full = lambda a, b: pl.BlockSpec((a, b), lambda i: (0, 0))
  row = lambda w: pl.BlockSpec((BM, w), lambda i: (i, 0))
  return pl.pallas_call(
      body,
      grid=(Mp // BM,),
      in_specs=[
          row(H), row(H), full(H, H), full(1, H),
          full(2 * H, 2 * H), full(1, 2 * H), full(2 * H, 2 * H),
          full(1, 2 * H), full(2 * H, 2 * H), full(1, 2 * H),
          full(2 * H, 2 * H), full(1, 2 * H), full(2 * H, H),
      ],
      out_specs=[row(4 * H), row(2 * H), row(2 * H), row(2 * H)],
      out_shape=[jax.ShapeDtypeStruct((Mp, 4 * H), jnp.float32),
                 jax.ShapeDtypeStruct((Mp, 2 * H), jnp.float32),
                 jax.ShapeDtypeStruct((Mp, 2 * H), jnp.float32),
                 jax.ShapeDtypeStruct((Mp, 2 * H), jnp.float32)],
  )(on, agg, wg, bg, wq, bq, wk, bk, wv, bv, ws, bs, wet)


def _out_call(numer, skip, o, cs, we, wlin, blin):
  def body(nu_ref, sk_ref, o_ref, cs_ref, we_r, wl_r, bl_r, om_ref):
    nu = nu_ref[...]
    we = we_r[...]
    s0 = 1.0 / (nu[:, 256:257] + 1e-16)
    s1 = 1.0 / (nu[:, 257:258] + 1e-16)
    o0 = (nu[:, 0:64] + jnp.dot(nu[:, 128:192], we[:, :H],
                                preferred_element_type=jnp.float32)) * s0
    o1 = (nu[:, 64:128] + jnp.dot(nu[:, 192:256], we[:, H:],
                                  preferred_element_type=jnp.float32)) * s1
    out = jnp.concatenate([o0, o1], 1) + sk_ref[...]
    l_h = jnp.dot(out, wl_r[...], preferred_element_type=jnp.float32) + bl_r[...]
    cs = cs_ref[...]
    om_ref[...] = o_ref[...] + l_h * cs[:, :H] + cs[:, H:]

  full = lambda a, b: pl.BlockSpec((a, b), lambda i: (0, 0))
  row = lambda w: pl.BlockSpec((BM, w), lambda i: (i, 0))
  return pl.pallas_call(
      body,
      grid=(Mp // BM,),
      in_specs=[row(NU), row(2 * H), row(H), row(2 * H),
                full(H, 2 * H), full(2 * H, H), full(1, H)],
      out_specs=row(H),
      out_shape=jax.ShapeDtypeStruct((Mp, H), jnp.float32),
  )(numer, skip, o, cs, we, wlin, blin)


def _ff_call(o, oh, stats, invn, w1, b1, w2, b2):
  def body(o_ref, oh_ref, st_ref, in_ref, w1_r, b1_r, w2_r, b2_r, on_ref):
    st = st_ref[...]
    invn = in_ref[...][:, 0:1]
    mean = st[:, 0:1] * invn
    var = st[:, 1:2] * invn - mean * mean
    rstd = jax.lax.rsqrt(var + EPS)
    mv = jnp.concatenate([mean, rstd, jnp.zeros((G, 14), jnp.float32)], 1)
    rows = jnp.dot(oh_ref[...], mv, preferred_element_type=jnp.float32)
    ob = o_ref[...]
    on = (ob - rows[:, 0:1]) * rows[:, 1:2]
    h = _lrelu(jnp.dot(on, w1_r[...], preferred_element_type=jnp.float32)
               + b1_r[...])
    on_ref[...] = ob + jnp.dot(h, w2_r[...],
                               preferred_element_type=jnp.float32) + b2_r[...]

  full = lambda a, b: pl.BlockSpec((a, b), lambda i: (0, 0))
  row = lambda w: pl.BlockSpec((BM, w), lambda i: (i, 0))
  return pl.pallas_call(
      body,
      grid=(Mp // BM,),
      in_specs=[row(H), row(G), full(G, 16), full(G, 8),
                full(H, 4 * H), full(1, 4 * H), full(4 * H, H), full(1, H)],
      out_specs=row(H),
      out_shape=jax.ShapeDtypeStruct((Mp, H), jnp.float32),
  )(o, oh, stats, invn, w1, b1, w2, b2)


def _pool_call(o, ohp, invc):
  def body(o_ref, oh_ref, ic_ref, p_ref):
    @pl.when(pl.program_id(0) == 0)
    def _():
      p_ref[...] = jnp.zeros_like(p_ref)
    p_ref[...] += jnp.dot(oh_ref[...].T, o_ref[...],
                          preferred_element_type=jnp.float32)

    @pl.when(pl.program_id(0) == Mp // BM - 1)
    def _():
      p_ref[...] *= ic_ref[...][:, 0:1]

  return pl.pallas_call(
      body,
      grid=(Mp // BM,),
      in_specs=[pl.BlockSpec((BM, H), lambda i: (i, 0)),
                pl.BlockSpec((BM, G), lambda i: (i, 0)),
                pl.BlockSpec((G, 8), lambda i: (0, 0))],
      out_specs=pl.BlockSpec((G, H), lambda i: (0, 0)),
      out_shape=jax.ShapeDtypeStruct((G, H), jnp.float32),
  )(o, ohp, invc)


def _pad_rows(a, rows):
  return jnp.pad(a, ((0, rows - a.shape[0]), (0, 0)))


def _worker_bnds(starts, ends):
  nr = starts.shape[0]
  e0 = (starts // CH) * CH
  ln = ends - e0
  s_ids = jnp.arange(NS, dtype=jnp.int32)
  nk = jnp.maximum(
      0, (ln[:, None] - s_ids[None, :] * CH + (NS * CH - 1)) // (NS * CH))
  rows = jnp.zeros((nr, NS, 2, 16), jnp.int32)
  rows = rows.at[:, :, :, 0].set(e0[:, None, None])
  rows = rows.at[:, :, :, 1].set(nk[:, :, None])
  return rows.reshape(nr * 32, 16)


def kernel(x, edge_index, edge_attr, batch, cond, params):
  batch = batch.astype(jnp.int32)
  ei = edge_index.astype(jnp.int32)

  u = jnp.arange(N, dtype=jnp.int32)
  v = batch + N
  sl = jnp.arange(M, dtype=jnp.int32)
  src0 = jnp.concatenate([ei[0], u, v])
  dst0 = jnp.concatenate([ei[1], v, u])
  src_a = jnp.concatenate([src0, sl])
  dst_a = jnp.concatenate([dst0, sl])
  perm = jnp.argsort(dst_a)
  dst_s = dst_a[perm]
  src_s = src_a[perm]
  pad_e = Eap - Ea
  dst_sp = jnp.concatenate([dst_s, jnp.full((pad_e,), 2 * Mp, jnp.int32)])
  src_sp = jnp.concatenate([src_s, jnp.zeros((pad_e,), jnp.int32)])
  perm_p = jnp.concatenate([perm, jnp.zeros((pad_e,), jnp.int32)])
  dst0_p = jnp.concatenate([dst0, jnp.full((E2np - E2n,), 2 * Mp, jnp.int32)])

  st4 = jnp.searchsorted(
      dst_sp, (jnp.arange(4, dtype=jnp.int32) * Q4)).astype(jnp.int32)
  en4 = jnp.concatenate([st4[1:], jnp.array([Eap], jnp.int32)])
  bnds4 = _worker_bnds(st4, en4)
  st16 = jnp.searchsorted(
      dst_sp, (jnp.arange(16, dtype=jnp.int32) * R16)).astype(jnp.int32)
  en16 = jnp.concatenate([st16[1:], jnp.array([Eap], jnp.int32)])
  bnds16 = _worker_bnds(st16, en16)

  deg_edges = jnp.searchsorted(
      dst_sp, jnp.arange(Mp + 1, dtype=jnp.int32)).astype(jnp.int32)
  cnt0 = (deg_edges[1:] - deg_edges[:-1] - 1).astype(jnp.float32)
  inv_cnt0 = 1.0 / jnp.clip(cnt0, 1.0)
  inv0_16 = jnp.zeros((Mp, 8), jnp.float32).at[:, 0].set(inv_cnt0)

  ab = jnp.concatenate([batch, jnp.arange(G, dtype=jnp.int32)])
  ab_p = jnp.concatenate([ab, jnp.full((Mp - M,), 2 * G, jnp.int32)])
  gcols = jnp.arange(G, dtype=jnp.int32)
  oh = (ab_p[:, None] == gcols[None, :]).astype(jnp.float32)
  ab_pool = jnp.where(jnp.arange(Mp) < N, ab_p, 2 * G)
  ohp = (ab_pool[:, None] == gcols[None, :]).astype(jnp.float32)
  cnt_b = jnp.searchsorted(batch, jnp.arange(G + 1, dtype=jnp.int32))
  cnt_g = (cnt_b[1:] - cnt_b[:-1]).astype(jnp.float32)
  invn = jnp.zeros((G, 8), jnp.float32).at[:, 0].set(
      1.0 / ((cnt_g + 1.0) * H))
  invc = jnp.zeros((G, 8), jnp.float32).at[:, 0].set(
      1.0 / jnp.clip(cnt_g, 1.0))

  ox = _mlp3_call(_pad_rows(x, Np), params["x2h"], Np)[:N]
  e_h = _mlp3_call(edge_attr, params["e2h"], E)
  c_h = _mlp3_call(jnp.pad(cond, ((0, BM - G), (0, 0))), params["c2h"], BM)[:G]

  e_p = jnp.zeros((2 * N, H), jnp.float32).at[:, 0].set(1.0)
  ae0 = jnp.concatenate([e_h, e_p])
  ae0_p = jnp.concatenate([ae0, jnp.zeros((E2np - E2n, H), jnp.float32)])
  sums = _sc_loop_attr(ae0_p, dst0_p)
  loop_attr = _scale_rows_call(sums, inv0_16)[:M]
  ae_full = jnp.concatenate([ae0, loop_attr])
  ae_s = _sc_permute_ae(ae_full, perm_p)

  o = _pad_rows(jnp.concatenate([ox, c_h]), Mp)

  for lp in params["layers"]:
    we = lp["e"]["w"]
    wet = jnp.concatenate([we[:, :H].T, we[:, H:].T], 0)
    st = _stats_call(o, oh)
    on, cs = _pre_call(o, oh, st, invn, c_h, lp["cscale"]["w"],
                       lp["cscale"]["b"].reshape(1, 2 * H))
    agg = _sc_gen(on, ae_s, src_sp, dst_sp, bnds4)
    qu, kt, vt, sk = _qkv_call(on, agg, lp, wet)
    qd_s, ks_s = _sc_gather_qk(qu, kt, src_sp, dst_sp)
    zb = _logits_call(qd_s, ks_s, ae_s)
    numer = _sc_numer(vt, ae_s, zb, src_sp, dst_sp, bnds16)
    o = _out_call(numer, sk, o, cs, we, lp["lin"]["w"],
                  lp["lin"]["b"].reshape(1, H))
    st2 = _stats_call(o, oh)
    o = _ff_call(o, oh, st2, invn, lp["ff"][0]["w"],
                 lp["ff"][0]["b"].reshape(1, 4 * H), lp["ff"][1]["w"],
                 lp["ff"][1]["b"].reshape(1, H))

  o_final = o[:N]
  pooled = _pool_call(o, ohp, invc)
  glob = jnp.concatenate([pooled, o[N:N + G]], 1)
  return o_final, glob


# speedup vs baseline: 9.9876x; 1.1375x over previous
import functools
import jax
import jax.numpy as jnp
import numpy as np
from jax import lax
from jax.experimental import pallas as pl
from jax.experimental.pallas import tpu as pltpu, tpu_sc as plsc

N = 50000
E = 800000
G = 128
XD = 128
ED = 16
GD = 32
H = 64
NH = 2
EPS = 1e-5

M = N + G
Mp = 50176
HM = Mp // 2
Ea = E + 2 * N + M
CH = 128
NS = 16
CHGRP = NS * CH * 2
Eap = ((Ea + CHGRP - 1) // CHGRP) * CHGRP
E2n = E + 2 * N
E2np = ((E2n + CHGRP - 1) // CHGRP) * CHGRP
Q4 = Mp // 4
R16 = Mp // 16
NU = 144
Np = 50048
BM = 256

_mesh = plsc.VectorSubcoreMesh(core_axis_name="c", subcore_axis_name="s")
_sc_params = pltpu.CompilerParams(use_tc_tiling_on_sc=False)


def _wid():
  return lax.axis_index("s") * 2 + lax.axis_index("c")


def _zero_buf(buf, rows, width):
  def zrow(r, _):
    for j in range(width // 16):
      buf[r, pl.ds(j * 16, 16)] = jnp.zeros((16,), jnp.float32)
    return 0
  lax.fori_loop(0, rows, zrow, 0)


def _zero_acc_stripe(acc, buf, s, half_rows, strip, nstrip, width):
  _zero_buf(buf, strip, width)
  per = half_rows // NS
  for t in range(nstrip):
    pltpu.sync_copy(buf, acc.at[pl.ds(s * per + t * strip, strip)])

  @pl.when(s == 0)
  def _():
    pltpu.sync_copy(buf.at[pl.ds(0, 8)], acc.at[pl.ds(half_rows, 8)])


def _local_idx(dst_v, dstl_v, base, nrows):
  for j in range(CH // 16):
    d = dst_v[pl.ds(j * 16, 16)]
    l = d - base
    oob = (l < 0) | (l >= nrows)
    dstl_v[pl.ds(j * 16, 16)] = jnp.where(oob, nrows, l)


@functools.partial(
    pl.kernel,
    out_type=jax.ShapeDtypeStruct((Mp, H), jnp.float32),
    mesh=_mesh, compiler_params=_sc_params,
    scratch_types=[
        pltpu.VMEM((CH,), jnp.int32),
        pltpu.VMEM((CH,), jnp.int32),
        pltpu.VMEM((CH, H), jnp.float32),
        pltpu.VMEM((112, H), jnp.float32),
        pltpu.VMEM_SHARED((HM + 8, H), jnp.float32),
        pltpu.SemaphoreType.DMA,
    ],
)
def _sc_loop_attr(ae0, dst0, out, dst_v, dstl_v, ae_v, buf_v, acc, sem):
  del sem
  c = lax.axis_index("c")
  s = lax.axis_index("s")
  base = c * HM
  _zero_acc_stripe(acc, buf_v, s, HM, 112, 14, H)
  plsc.subcore_barrier()

  def body(k, _):
    eb = (k * NS + s) * CH
    pltpu.sync_copy(dst0.at[pl.ds(eb, CH)], dst_v)
    pltpu.sync_copy(ae0.at[pl.ds(eb, CH)], ae_v)
    _local_idx(dst_v, dstl_v, base, HM)
    pltpu.sync_copy(ae_v, acc.at[dstl_v], add=True)
    return 0

  lax.fori_loop(0, E2np // CHGRP * 2, body, 0)
  plsc.subcore_barrier()
  per = HM // NS
  for t in range(14):
    pltpu.sync_copy(acc.at[pl.ds(s * per + t * 112, 112)], buf_v)
    pltpu.sync_copy(buf_v, out.at[pl.ds(base + s * per + t * 112, 112)])


@functools.partial(
    pl.kernel,
    out_type=jax.ShapeDtypeStruct((Eap, H), jnp.float32),
    mesh=_mesh, compiler_params=_sc_params,
    scratch_types=[
        pltpu.VMEM((CH,), jnp.int32),
        pltpu.VMEM((CH, H), jnp.float32),
        pltpu.SemaphoreType.DMA,
    ],
)
def _sc_permute_ae(ae_full, perm, out, idx_v, rows_v, sem):
  c = lax.axis_index("c")
  s = lax.axis_index("s")

  def body(k, _):
    eb = c * (Eap // 2) + (k * NS + s) * CH
    pltpu.sync_copy(perm.at[pl.ds(eb, CH)], idx_v)
    pltpu.async_copy(ae_full.at[idx_v], rows_v, sem).wait()
    pltpu.sync_copy(rows_v, out.at[pl.ds(eb, CH)])
    return 0

  lax.fori_loop(0, Eap // CHGRP, body, 0)


@functools.partial(
    pl.kernel,
    out_type=jax.ShapeDtypeStruct((Mp, H), jnp.float32),
    mesh=_mesh, compiler_params=_sc_params,
    scratch_types=[
        pltpu.VMEM((16,), jnp.int32),
        pltpu.VMEM((CH,), jnp.int32),
        pltpu.VMEM((CH,), jnp.int32),
        pltpu.VMEM((CH,), jnp.int32),
        pltpu.VMEM((CH, H), jnp.float32),
        pltpu.VMEM((CH, H), jnp.float32),
        pltpu.VMEM((CH, H), jnp.float32),
        pltpu.VMEM((112, H), jnp.float32),
        pltpu.VMEM_SHARED((Q4 + 8, H), jnp.float32),
        pltpu.SemaphoreType.DMA,
    ],
)
def _sc_gen(on_t, ae_s, src_s, dst_s, bnds, out, bnds_v, src_v, dst_v, dstl_v,
            rows_v, ae_v, msg_v, buf_v, acc, sem):
  c = lax.axis_index("c")
  s = lax.axis_index("s")
  wid = _wid()
  for rr in range(2):
    rg = c * 2 + rr
    base = rg * Q4
    _zero_acc_stripe(acc, buf_v, s, Q4, 112, 7, H)
    plsc.subcore_barrier()

    pltpu.sync_copy(bnds.at[rg * 32 + wid], bnds_v)
    bv = bnds_v[...]
    e0 = bv[0]
    nk = bv[1]

    def body(k, _):
      eb = pl.multiple_of(e0 + (k * NS + s) * CH, 8)
      pltpu.sync_copy(src_s.at[pl.ds(eb, CH)], src_v)
      pltpu.sync_copy(dst_s.at[pl.ds(eb, CH)], dst_v)
      pltpu.async_copy(on_t.at[src_v], rows_v, sem).wait()
      pltpu.sync_copy(ae_s.at[pl.ds(eb, CH)], ae_v)

      def crow(r, _):
        for t in range(H // 16):
          v = rows_v[r, pl.ds(t * 16, 16)] + ae_v[r, pl.ds(t * 16, 16)]
          msg_v[r, pl.ds(t * 16, 16)] = jnp.maximum(v, 0.0) + 1e-7
        return 0
      lax.fori_loop(0, CH, crow, 0)

      _local_idx(dst_v, dstl_v, base, Q4)
      pltpu.sync_copy(msg_v, acc.at[dstl_v], add=True)
      return 0

    lax.fori_loop(0, nk, body, 0)
    plsc.subcore_barrier()
    per = Q4 // NS
    for t in range(7):
      pltpu.sync_copy(acc.at[pl.ds(s * per + t * 112, 112)], buf_v)
      pltpu.sync_copy(buf_v, out.at[pl.ds(base + s * per + t * 112, 112)])
    plsc.subcore_barrier()


@functools.partial(
    pl.kernel,
    out_type=jax.ShapeDtypeStruct((Eap, 32), jnp.float32),
    mesh=_mesh, compiler_params=_sc_params,
    scratch_types=[
        pltpu.VMEM((CH,), jnp.int32),
        pltpu.VMEM((CH,), jnp.int32),
        pltpu.VMEM((CH,), jnp.int32),
        pltpu.VMEM((CH, 2 * H), jnp.float32),
        pltpu.VMEM((CH, 2 * H), jnp.float32),
        pltpu.VMEM((CH, 2 * H), jnp.float32),
        pltpu.VMEM((CH, 32), jnp.float32),
        pltpu.SemaphoreType.DMA,
        pltpu.SemaphoreType.DMA,
    ],
)
def _sc_logits_part(q_t, k_t, eev_s, src_s, dst_s, p_out, src_v, dst_v,
                    dstc_v, q_v, k_v, eev_v, p_v, sem1, sem2):
  c = lax.axis_index("c")
  s = lax.axis_index("s")

  def body(k, _):
    eb = c * (Eap // 2) + (k * NS + s) * CH
    pltpu.sync_copy(src_s.at[pl.ds(eb, CH)], src_v)
    pltpu.sync_copy(dst_s.at[pl.ds(eb, CH)], dst_v)
    for j in range(CH // 16):
      d = dst_v[pl.ds(j * 16, 16)]
      dstc_v[pl.ds(j * 16, 16)] = jnp.where(d >= Mp, 0, d)
    cp1 = pltpu.async_copy(q_t.at[dstc_v], q_v, sem1)
    cp2 = pltpu.async_copy(k_t.at[src_v], k_v, sem2)
    pltpu.sync_copy(eev_s.at[pl.ds(eb, CH)], eev_v)
    cp1.wait()
    cp2.wait()

    def crow(r, _):
      acc0 = jnp.zeros((16,), jnp.float32)
      acc1 = jnp.zeros((16,), jnp.float32)
      for t in range(H // 16):
        acc0 += q_v[r, pl.ds(t * 16, 16)] * (
            k_v[r, pl.ds(t * 16, 16)] + eev_v[r, pl.ds(t * 16, 16)])
        acc1 += q_v[r, pl.ds(64 + t * 16, 16)] * (
            k_v[r, pl.ds(64 + t * 16, 16)] + eev_v[r, pl.ds(64 + t * 16, 16)])
      p_v[r, pl.ds(0, 16)] = acc0
      p_v[r, pl.ds(16, 16)] = acc1
      return 0

    lax.fori_loop(0, CH, crow, 0)
    pltpu.sync_copy(p_v, p_out.at[pl.ds(eb, CH)])
    return 0

  lax.fori_loop(0, Eap // CHGRP, body, 0)


BE = 512


def _ee_call(ae, we):
  def body(ae_ref, we_ref, o_ref):
    o_ref[...] = jnp.dot(ae_ref[...], we_ref[...],
                         preferred_element_type=jnp.float32)

  return pl.pallas_call(
      body,
      grid=(Eap // BE,),
      in_specs=[pl.BlockSpec((BE, H), lambda i: (i, 0)),
                pl.BlockSpec((H, 2 * H), lambda i: (0, 0))],
      out_specs=pl.BlockSpec((BE, 2 * H), lambda i: (i, 0)),
      out_shape=jax.ShapeDtypeStruct((Eap, 2 * H), jnp.float32),
  )(ae, we)


def _logits_call(p):
  def body(p_ref, z_ref):
    p_b = p_ref[...]
    l0 = jnp.sum(p_b[:, :16], axis=1, keepdims=True)
    l1 = jnp.sum(p_b[:, 16:], axis=1, keepdims=True)
    ones = jnp.ones((1, 16), jnp.float32)
    z_ref[...] = jnp.concatenate(
        [jnp.exp(l0 * 0.125) * ones, jnp.exp(l1 * 0.125) * ones], 1)

  return pl.pallas_call(
      body,
      grid=(Eap // BE,),
      in_specs=[pl.BlockSpec((BE, 32), lambda i: (i, 0))],
      out_specs=pl.BlockSpec((BE, 32), lambda i: (i, 0)),
      out_shape=jax.ShapeDtypeStruct((Eap, 32), jnp.float32),
  )(p)


@functools.partial(
    pl.kernel,
    out_type=jax.ShapeDtypeStruct((Mp, NU), jnp.float32),
    mesh=_mesh, compiler_params=_sc_params,
    scratch_types=[
        pltpu.VMEM((16,), jnp.int32),
        pltpu.VMEM((CH,), jnp.int32),
        pltpu.VMEM((CH,), jnp.int32),
        pltpu.VMEM((CH,), jnp.int32),
        pltpu.VMEM((CH, 2 * H), jnp.float32),
        pltpu.VMEM((CH, 2 * H), jnp.float32),
        pltpu.VMEM((CH, 32), jnp.float32),
        pltpu.VMEM((CH, NU), jnp.float32),
        pltpu.VMEM((28, NU), jnp.float32),
        pltpu.VMEM_SHARED((R16 + 8, NU), jnp.float32),
        pltpu.SemaphoreType.DMA,
    ],
)
def _sc_numer(v_t, eev_s, z, src_s, dst_s, bnds, out, bnds_v, src_v, dst_v,
              dstl_v, v_v, eev_v, z_v, row_v, buf_v, acc, sem):
  c = lax.axis_index("c")
  s = lax.axis_index("s")
  lanes = lax.iota(jnp.int32, 16)
  wid = _wid()

  for rr in range(8):
    rg = c * 8 + rr
    base = rg * R16
    _zero_acc_stripe(acc, buf_v, s, R16, 28, 7, NU)
    plsc.subcore_barrier()

    pltpu.sync_copy(bnds.at[rg * 32 + wid], bnds_v)
    bv = bnds_v[...]
    e0 = bv[0]
    nk = bv[1]

    def body(k, _):
      eb = pl.multiple_of(e0 + (k * NS + s) * CH, 8)
      pltpu.sync_copy(src_s.at[pl.ds(eb, CH)], src_v)
      pltpu.sync_copy(dst_s.at[pl.ds(eb, CH)], dst_v)
      pltpu.async_copy(v_t.at[src_v], v_v, sem).wait()
      pltpu.sync_copy(eev_s.at[pl.ds(eb, CH)], eev_v)
      pltpu.sync_copy(z.at[pl.ds(eb, CH)], z_v)

      def crow(r, _):
        zb0 = z_v[r, pl.ds(0, 16)]
        zb1 = z_v[r, pl.ds(16, 16)]
        for t in range(H // 16):
          row_v[r, pl.ds(t * 16, 16)] = zb0 * (
              v_v[r, pl.ds(t * 16, 16)] + eev_v[r, pl.ds(t * 16, 16)])
          row_v[r, pl.ds(64 + t * 16, 16)] = zb1 * (
              v_v[r, pl.ds(64 + t * 16, 16)] + eev_v[r, pl.ds(64 + t * 16, 16)])
        row_v[r, pl.ds(128, 16)] = jnp.where(
            lanes == 0, zb0, jnp.where(lanes == 1, zb1, 0.0))
        return 0

      lax.fori_loop(0, CH, crow, 0)
      _local_idx(dst_v, dstl_v, base, R16)
      pltpu.sync_copy(row_v, acc.at[dstl_v], add=True)
      return 0

    lax.fori_loop(0, nk, body, 0)
    plsc.subcore_barrier()
    per = R16 // NS
    for t in range(7):
      pltpu.sync_copy(acc.at[pl.ds(s * per + t * 28, 28)], buf_v)
      pltpu.sync_copy(buf_v, out.at[pl.ds(base + s * per + t * 28, 28)])
    plsc.subcore_barrier()


def _lrelu(x):
  return jnp.where(x >= 0, x, 0.01 * x)


def _mlp3_call(x, ps, rows):
  din = x.shape[1]

  def body(x_ref, w1, b1, w2, b2, w3, b3, o_ref):
    h = _lrelu(jnp.dot(x_ref[...], w1[...],
                       preferred_element_type=jnp.float32) + b1[...])
    h = _lrelu(jnp.dot(h, w2[...], preferred_element_type=jnp.float32) + b2[...])
    o_ref[...] = jnp.dot(h, w3[...], preferred_element_type=jnp.float32) + b3[...]

  full = lambda a, b: pl.BlockSpec((a, b), lambda i: (0, 0))
  return pl.pallas_call(
      body,
      grid=(rows // BM,),
      in_specs=[
          pl.BlockSpec((BM, din), lambda i: (i, 0)),
          full(din, H), full(1, H), full(H, H), full(1, H), full(H, H),
          full(1, H),
      ],
      out_specs=pl.BlockSpec((BM, H), lambda i: (i, 0)),
      out_shape=jax.ShapeDtypeStruct((rows, H), jnp.float32),
  )(x, ps[0]["w"], ps[0]["b"].reshape(1, H), ps[1]["w"],
    ps[1]["b"].reshape(1, H), ps[2]["w"], ps[2]["b"].reshape(1, H))


def _scale_rows_call(sums, inv):
  def body(s_ref, i_ref, o_ref):
    o_ref[...] = s_ref[...] * i_ref[...][:, 0:1]

  return pl.pallas_call(
      body,
      grid=(Mp // BM,),
      in_specs=[pl.BlockSpec((BM, H), lambda i: (i, 0)),
                pl.BlockSpec((BM, 8), lambda i: (i, 0))],
      out_specs=pl.BlockSpec((BM, H), lambda i: (i, 0)),
      out_shape=jax.ShapeDtypeStruct((Mp, H), jnp.float32),
  )(sums, inv)


def _stats_call(o, oh):
  def body(o_ref, oh_ref, s_ref):
    @pl.when(pl.program_id(0) == 0)
    def _():
      s_ref[...] = jnp.zeros_like(s_ref)
    x = o_ref[...]
    s1 = jnp.sum(x, axis=1, keepdims=True)
    s2 = jnp.sum(x * x, axis=1, keepdims=True)
    both = jnp.concatenate([s1, s2, jnp.zeros((BM, 14), jnp.float32)], 1)
    s_ref[...] += jnp.dot(oh_ref[...].T, both,
                          preferred_element_type=jnp.float32)

  return pl.pallas_call(
      body,
      grid=(Mp // BM,),
      in_specs=[pl.BlockSpec((BM, H), lambda i: (i, 0)),
                pl.BlockSpec((BM, G), lambda i: (i, 0))],
      out_specs=pl.BlockSpec((G, 16), lambda i: (0, 0)),
      out_shape=jax.ShapeDtypeStruct((G, 16), jnp.float32),
  )(o, oh)


def _pre_call(o, oh, stats, invn, c_h, wcs, bcs):
  def body(o_ref, oh_ref, st_ref, in_ref, c_ref, w_ref, b_ref, on_ref, cs_ref):
    st = st_ref[...]
    invn = in_ref[...][:, 0:1]
    mean = st[:, 0:1] * invn
    var = st[:, 1:2] * invn - mean * mean
    rstd = jax.lax.rsqrt(var + EPS)
    mv = jnp.concatenate([mean, rstd, jnp.zeros((G, 14), jnp.float32)], 1)
    rows = jnp.dot(oh_ref[...], mv, preferred_element_type=jnp.float32)
    on_ref[...] = (o_ref[...] - rows[:, 0:1]) * rows[:, 1:2]
    csc = jnp.dot(c_ref[...], w_ref[...],
                  preferred_element_type=jnp.float32) + b_ref[...]
    cs_ref[...] = jnp.dot(oh_ref[...], csc, preferred_element_type=jnp.float32)

  full = lambda a, b: pl.BlockSpec((a, b), lambda i: (0, 0))
  return pl.pallas_call(
      body,
      grid=(Mp // BM,),
      in_specs=[
          pl.BlockSpec((BM, H), lambda i: (i, 0)),
          pl.BlockSpec((BM, G), lambda i: (i, 0)),
          full(G, 16), full(G, 8), full(G, H), full(H, 2 * H), full(1, 2 * H),
      ],
      out_specs=[pl.BlockSpec((BM, H), lambda i: (i, 0)),
                 pl.BlockSpec((BM, 2 * H), lambda i: (i, 0))],
      out_shape=[jax.ShapeDtypeStruct((Mp, H), jnp.float32),
                 jax.ShapeDtypeStruct((Mp, 2 * H), jnp.float32)],
  )(o, oh, stats, invn, c_h, wcs, bcs)


def _qkv_call(on, agg, lp):
  wg, bg = lp["gen_mlp"]["w"], lp["gen_mlp"]["b"].reshape(1, H)
  wq, bq = lp["q"]["w"], lp["q"]["b"].reshape(1, 2 * H)
  wk, bk = lp["k"]["w"], lp["k"]["b"].reshape(1, 2 * H)
  wv, bv = lp["v"]["w"], lp["v"]["b"].reshape(1, 2 * H)
  ws, bs = lp["skip"]["w"], lp["skip"]["b"].reshape(1, 2 * H)

  def body(on_ref, ag_ref, wg_r, bg_r, wq_r, bq_r, wk_r, bk_r, wv_r, bv_r,
           ws_r, bs_r, q_ref, k_ref, v_ref, sk_ref):
    on_b = on_ref[...]
    ag2 = jnp.dot(ag_ref[...] + on_b, wg_r[...],
                  preferred_element_type=jnp.float32) + bg_r[...]

    def two(w, b):
      return (jnp.dot(on_b, w[:H], preferred_element_type=jnp.float32)
              + jnp.dot(ag2, w[H:], preferred_element_type=jnp.float32) + b)

    q_ref[...] = two(wq_r[...], bq_r[...])
    k_ref[...] = two(wk_r[...], bk_r[...])
    v_ref[...] = two(wv_r[...], bv_r[...])
    sk_ref[...] = two(ws_r[...], bs_r[...])

  full = lambda a, b: pl.BlockSpec((a, b), lambda i: (0, 0))
  row = lambda w: pl.BlockSpec((BM, w), lambda i: (i, 0))
  return pl.pallas_call(
      body,
      grid=(Mp // BM,),
      in_specs=[
          row(H), row(H), full(H, H), full(1, H),
          full(2 * H, 2 * H), full(1, 2 * H), full(2 * H, 2 * H),
          full(1, 2 * H), full(2 * H, 2 * H), full(1, 2 * H),
          full(2 * H, 2 * H), full(1, 2 * H),
      ],
      out_specs=[row(2 * H), row(2 * H), row(2 * H), row(2 * H)],
      out_shape=[jax.ShapeDtypeStruct((Mp, 2 * H), jnp.float32),
                 jax.ShapeDtypeStruct((Mp, 2 * H), jnp.float32),
                 jax.ShapeDtypeStruct((Mp, 2 * H), jnp.float32),
                 jax.ShapeDtypeStruct((Mp, 2 * H), jnp.float32)],
  )(on, agg, wg, bg, wq, bq, wk, bk, wv, bv, ws, bs)


def _out_call(numer, skip, o, cs, wlin, blin):
  def body(nu_ref, sk_ref, o_ref, cs_ref, wl_r, bl_r, om_ref):
    nu = nu_ref[...]
    s0 = 1.0 / (nu[:, 128:129] + 1e-16)
    s1 = 1.0 / (nu[:, 129:130] + 1e-16)
    out = jnp.concatenate([nu[:, 0:64] * s0, nu[:, 64:128] * s1], 1) + sk_ref[...]
    l_h = jnp.dot(out, wl_r[...], preferred_element_type=jnp.float32) + bl_r[...]
    cs = cs_ref[...]
    om_ref[...] = o_ref[...] + l_h * cs[:, :H] + cs[:, H:]

  full = lambda a, b: pl.BlockSpec((a, b), lambda i: (0, 0))
  row = lambda w: pl.BlockSpec((BM, w), lambda i: (i, 0))
  return pl.pallas_call(
      body,
      grid=(Mp // BM,),
      in_specs=[row(NU), row(2 * H), row(H), row(2 * H),
                full(2 * H, H), full(1, H)],
      out_specs=row(H),
      out_shape=jax.ShapeDtypeStruct((Mp, H), jnp.float32),
  )(numer, skip, o, cs, wlin, blin)


def _ff_call(o, oh, stats, invn, w1, b1, w2, b2):
  def body(o_ref, oh_ref, st_ref, in_ref, w1_r, b1_r, w2_r, b2_r, on_ref):
    st = st_ref[...]
    invn = in_ref[...][:, 0:1]
    mean = st[:, 0:1] * invn
    var = st[:, 1:2] * invn - mean * mean
    rstd = jax.lax.rsqrt(var + EPS)
    mv = jnp.concatenate([mean, rstd, jnp.zeros((G, 14), jnp.float32)], 1)
    rows = jnp.dot(oh_ref[...], mv, preferred_element_type=jnp.float32)
    ob = o_ref[...]
    on = (ob - rows[:, 0:1]) * rows[:, 1:2]
    h = _lrelu(jnp.dot(on, w1_r[...], preferred_element_type=jnp.float32)
               + b1_r[...])
    on_ref[...] = ob + jnp.dot(h, w2_r[...],
                               preferred_element_type=jnp.float32) + b2_r[...]

  full = lambda a, b: pl.BlockSpec((a, b), lambda i: (0, 0))
  row = lambda w: pl.BlockSpec((BM, w), lambda i: (i, 0))
  return pl.pallas_call(
      body,
      grid=(Mp // BM,),
      in_specs=[row(H), row(G), full(G, 16), full(G, 8),
                full(H, 4 * H), full(1, 4 * H), full(4 * H, H), full(1, H)],
      out_specs=row(H),
      out_shape=jax.ShapeDtypeStruct((Mp, H), jnp.float32),
  )(o, oh, stats, invn, w1, b1, w2, b2)


def _pool_call(o, ohp, invc):
  def body(o_ref, oh_ref, ic_ref, p_ref):
    @pl.when(pl.program_id(0) == 0)
    def _():
      p_ref[...] = jnp.zeros_like(p_ref)
    p_ref[...] += jnp.dot(oh_ref[...].T, o_ref[...],
                          preferred_element_type=jnp.float32)

    @pl.when(pl.program_id(0) == Mp // BM - 1)
    def _():
      p_ref[...] *= ic_ref[...][:, 0:1]

  return pl.pallas_call(
      body,
      grid=(Mp // BM,),
      in_specs=[pl.BlockSpec((BM, H), lambda i: (i, 0)),
                pl.BlockSpec((BM, G), lambda i: (i, 0)),
                pl.BlockSpec((G, 8), lambda i: (0, 0))],
      out_specs=pl.BlockSpec((G, H), lambda i: (0, 0)),
      out_shape=jax.ShapeDtypeStruct((G, H), jnp.float32),
  )(o, ohp, invc)


def _pad_rows(a, rows):
  return jnp.pad(a, ((0, rows - a.shape[0]), (0, 0)))


def _worker_bnds(starts, ends):
  nr = starts.shape[0]
  e0 = (starts // CH) * CH
  ln = ends - e0
  s_ids = jnp.arange(NS, dtype=jnp.int32)
  nk = jnp.maximum(
      0, (ln[:, None] - s_ids[None, :] * CH + (NS * CH - 1)) // (NS * CH))
  rows = jnp.zeros((nr, NS, 2, 16), jnp.int32)
  rows = rows.at[:, :, :, 0].set(e0[:, None, None])
  rows = rows.at[:, :, :, 1].set(nk[:, :, None])
  return rows.reshape(nr * 32, 16)


def kernel(x, edge_index, edge_attr, batch, cond, params):
  batch = batch.astype(jnp.int32)
  ei = edge_index.astype(jnp.int32)

  u = jnp.arange(N, dtype=jnp.int32)
  v = batch + N
  sl = jnp.arange(M, dtype=jnp.int32)
  src0 = jnp.concatenate([ei[0], u, v])
  dst0 = jnp.concatenate([ei[1], v, u])
  src_a = jnp.concatenate([src0, sl])
  dst_a = jnp.concatenate([dst0, sl])
  perm = jnp.argsort(dst_a)
  dst_s = dst_a[perm]
  src_s = src_a[perm]
  pad_e = Eap - Ea
  dst_sp = jnp.concatenate([dst_s, jnp.full((pad_e,), 2 * Mp, jnp.int32)])
  src_sp = jnp.concatenate([src_s, jnp.zeros((pad_e,), jnp.int32)])
  perm_p = jnp.concatenate([perm, jnp.zeros((pad_e,), jnp.int32)])
  dst0_p = jnp.concatenate([dst0, jnp.full((E2np - E2n,), 2 * Mp, jnp.int32)])

  st4 = jnp.searchsorted(
      dst_sp, (jnp.arange(4, dtype=jnp.int32) * Q4)).astype(jnp.int32)
  en4 = jnp.concatenate([st4[1:], jnp.array([Eap], jnp.int32)])
  bnds4 = _worker_bnds(st4, en4)
  st16 = jnp.searchsorted(
      dst_sp, (jnp.arange(16, dtype=jnp.int32) * R16)).astype(jnp.int32)
  en16 = jnp.concatenate([st16[1:], jnp.array([Eap], jnp.int32)])
  bnds16 = _worker_bnds(st16, en16)

  deg_edges = jnp.searchsorted(
      dst_sp, jnp.arange(Mp + 1, dtype=jnp.int32)).astype(jnp.int32)
  cnt0 = (deg_edges[1:] - deg_edges[:-1] - 1).astype(jnp.float32)
  inv_cnt0 = 1.0 / jnp.clip(cnt0, 1.0)
  inv0_16 = jnp.zeros((Mp, 8), jnp.float32).at[:, 0].set(inv_cnt0)

  ab = jnp.concatenate([batch, jnp.arange(G, dtype=jnp.int32)])
  ab_p = jnp.concatenate([ab, jnp.full((Mp - M,), 2 * G, jnp.int32)])
  gcols = jnp.arange(G, dtype=jnp.int32)
  oh = (ab_p[:, None] == gcols[None, :]).astype(jnp.float32)
  ab_pool = jnp.where(jnp.arange(Mp) < N, ab_p, 2 * G)
  ohp = (ab_pool[:, None] == gcols[None, :]).astype(jnp.float32)
  cnt_b = jnp.searchsorted(batch, jnp.arange(G + 1, dtype=jnp.int32))
  cnt_g = (cnt_b[1:] - cnt_b[:-1]).astype(jnp.float32)
  invn = jnp.zeros((G, 8), jnp.float32).at[:, 0].set(
      1.0 / ((cnt_g + 1.0) * H))
  invc = jnp.zeros((G, 8), jnp.float32).at[:, 0].set(
      1.0 / jnp.clip(cnt_g, 1.0))

  ox = _mlp3_call(_pad_rows(x, Np), params["x2h"], Np)[:N]
  e_h = _mlp3_call(edge_attr, params["e2h"], E)
  c_h = _mlp3_call(jnp.pad(cond, ((0, BM - G), (0, 0))), params["c2h"], BM)[:G]

  e_p = jnp.zeros((2 * N, H), jnp.float32).at[:, 0].set(1.0)
  ae0 = jnp.concatenate([e_h, e_p])
  ae0_p = jnp.concatenate([ae0, jnp.zeros((E2np - E2n, H), jnp.float32)])
  sums = _sc_loop_attr(ae0_p, dst0_p)
  loop_attr = _scale_rows_call(sums, inv0_16)[:M]
  ae_full = jnp.concatenate([ae0, loop_attr])
  ae_s = _sc_permute_ae(ae_full, perm_p)

  o = _pad_rows(jnp.concatenate([ox, c_h]), Mp)

  for lp in params["layers"]:
    eev = _ee_call(ae_s, lp["e"]["w"])
    st = _stats_call(o, oh)
    on, cs = _pre_call(o, oh, st, invn, c_h, lp["cscale"]["w"],
                       lp["cscale"]["b"].reshape(1, 2 * H))
    agg = _sc_gen(on, ae_s, src_sp, dst_sp, bnds4)
    qt, kt, vt, sk = _qkv_call(on, agg, lp)
    pt = _sc_logits_part(qt, kt, eev, src_sp, dst_sp)
    zb = _logits_call(pt)
    numer = _sc_numer(vt, eev, zb, src_sp, dst_sp, bnds16)
    o = _out_call(numer, sk, o, cs, lp["lin"]["w"],
                  lp["lin"]["b"].reshape(1, H))
    st2 = _stats_call(o, oh)
    o = _ff_call(o, oh, st2, invn, lp["ff"][0]["w"],
                 lp["ff"][0]["b"].reshape(1, 4 * H), lp["ff"][1]["w"],
                 lp["ff"][1]["b"].reshape(1, H))

  o_final = o[:N]
  pooled = _pool_call(o, ohp, invc)
  glob = jnp.concatenate([pooled, o[N:N + G]], 1)
  return o_final, glob
